# Initial kernel scaffold; baseline (speedup 1.0000x reference)
import jax
import jax.numpy as jnp
from jax.experimental import pallas as pl


def kernel(boxes, scores):
    raise NotImplementedError("write your pallas kernel here")


# speedup vs baseline: 166.3338x; 166.3338x over previous
import functools

import jax
import jax.numpy as jnp
from jax import lax
from jax.experimental import pallas as pl
from jax.experimental.pallas import tpu as pltpu

_CNF = 0.5
_IOU = 0.5
_N = 5000
_B = 512
_NB = 10
_NPAD = _B * _NB


def _corners_tall(blk):
    cx, cy, w, h = blk[:, 0:1], blk[:, 1:2], blk[:, 2:3], blk[:, 3:4]
    x1, y1 = cx - w / 2.0, cy - h / 2.0
    x2, y2 = cx + w / 2.0, cy + h / 2.0
    return x1, y1, x2, y2, (x2 - x1) * (y2 - y1)


def _corners_wide(blk):
    cx, cy, w, h = blk[0:1, :], blk[1:2, :], blk[2:3, :], blk[3:4, :]
    x1, y1 = cx - w / 2.0, cy - h / 2.0
    x2, y2 = cx + w / 2.0, cy + h / 2.0
    return x1, y1, x2, y2, (x2 - x1) * (y2 - y1)


def _overlap(tall, wide):
    tx1, ty1, tx2, ty2, ta = tall
    wx1, wy1, wx2, wy2, wa = wide
    ix = jnp.maximum(0.0, jnp.minimum(tx2, wx2) - jnp.maximum(tx1, wx1))
    iy = jnp.maximum(0.0, jnp.minimum(ty2, wy2) - jnp.maximum(ty1, wy1))
    inter = ix * iy
    union = ta + wa - inter
    return jnp.where(inter > _IOU * (union + 1e-9), 1.0, 0.0)


def _nms_body(bt_ref, bw_ref, sw_ref, out_ref, keep_ref):
    keep_ref[...] = jnp.zeros((_NB, 1, _B), jnp.float32)

    count = jnp.sum((sw_ref[...] > _CNF).astype(jnp.int32))
    nb = (count + (_B - 1)) // _B

    row_lt_col = (
        lax.broadcasted_iota(jnp.int32, (_B, _B), 0)
        < lax.broadcasted_iota(jnp.int32, (_B, _B), 1)
    ).astype(jnp.float32)

    def outer(j, _):
        wide_j = _corners_wide(bw_ref[j])
        valid_j = (sw_ref[j] > _CNF).astype(jnp.float32)

        def cross(i, sup):
            o = _overlap(_corners_tall(bt_ref[i]), wide_j)
            return sup + jnp.dot(keep_ref[i], o,
                                 preferred_element_type=jnp.float32)

        sup = lax.fori_loop(0, j, cross, jnp.zeros((1, _B), jnp.float32))
        m0 = jnp.where(sup > 0.0, 0.0, valid_j)

        o_jj = _overlap(_corners_tall(bt_ref[j]), wide_j) * row_lt_col

        def cond(c):
            return c[1]

        def body(c):
            k, _ = c
            s = jnp.dot(k, o_jj, preferred_element_type=jnp.float32)
            k2 = jnp.where(s > 0.0, 0.0, m0)
            return k2, jnp.any(k2 != k)

        k_fix, _ = lax.while_loop(cond, body, (m0, True))
        keep_ref[j] = k_fix
        return 0

    lax.fori_loop(0, nb, outer, 0)

    for j in range(_NB):
        x1, y1, x2, y2, _ = _corners_wide(bw_ref[j])
        row = jnp.concatenate([x1, y1, x2, y2, sw_ref[j]], axis=0)
        out_ref[j] = row * keep_ref[j]


@jax.jit
def kernel(boxes, scores):
    order = jnp.argsort(-scores)
    sb = boxes[order]
    ss = scores[order]

    sb = jnp.concatenate(
        [sb, jnp.zeros((_NPAD - _N, 4), jnp.float32)], axis=0)
    ss = jnp.concatenate(
        [ss, jnp.full((_NPAD - _N,), -1.0, jnp.float32)], axis=0)

    bt = sb.reshape(_NB, _B, 4)
    bw = jnp.transpose(bt, (0, 2, 1))
    sw = ss.reshape(_NB, 1, _B)

    ow = pl.pallas_call(
        _nms_body,
        out_shape=jax.ShapeDtypeStruct((_NB, 5, _B), jnp.float32),
        scratch_shapes=[pltpu.VMEM((_NB, 1, _B), jnp.float32)],
    )(bt, bw, sw)

    out = jnp.transpose(ow, (1, 0, 2)).reshape(5, _NPAD)[:, :_N]
    return out.T


# speedup vs baseline: 334.9003x; 2.0134x over previous
import functools

import jax
import jax.numpy as jnp
from jax import lax
from jax.experimental import pallas as pl
from jax.experimental.pallas import tpu as pltpu

_CNF = 0.5
_IOU = 0.5
_N = 5000
_B = 512
_NB = 10
_NPAD = _B * _NB


def _corners_tall(blk):
    cx, cy, w, h = blk[:, 0:1], blk[:, 1:2], blk[:, 2:3], blk[:, 3:4]
    x1, y1 = cx - w / 2.0, cy - h / 2.0
    x2, y2 = cx + w / 2.0, cy + h / 2.0
    return x1, y1, x2, y2, (x2 - x1) * (y2 - y1)


def _corners_wide(blk):
    cx, cy, w, h = blk[0:1, :], blk[1:2, :], blk[2:3, :], blk[3:4, :]
    x1, y1 = cx - w / 2.0, cy - h / 2.0
    x2, y2 = cx + w / 2.0, cy + h / 2.0
    return x1, y1, x2, y2, (x2 - x1) * (y2 - y1)


def _overlap(tall, wide):
    tx1, ty1, tx2, ty2, ta = tall
    wx1, wy1, wx2, wy2, wa = wide
    ix = jnp.maximum(0.0, jnp.minimum(tx2, wx2) - jnp.maximum(tx1, wx1))
    iy = jnp.maximum(0.0, jnp.minimum(ty2, wy2) - jnp.maximum(ty1, wy1))
    inter = ix * iy
    union = ta + wa - inter
    return jnp.where(inter > _IOU * (union + 1e-9), 1.0, 0.0)


def _nms_body(bt_ref, bw_ref, sw_ref, out_ref, keep_ref):
    keep_ref[...] = jnp.zeros((_NB, 1, _B), jnp.float32)

    count = jnp.sum((sw_ref[...] > _CNF).astype(jnp.int32))
    nb = (count + (_B - 1)) // _B

    row_lt_col = (
        lax.broadcasted_iota(jnp.int32, (_B, _B), 0)
        < lax.broadcasted_iota(jnp.int32, (_B, _B), 1)
    ).astype(jnp.float32)

    def outer(j, _):
        wide_j = _corners_wide(bw_ref[j])
        valid_j = (sw_ref[j] > _CNF).astype(jnp.float32)

        def cross(i, sup):
            o = _overlap(_corners_tall(bt_ref[i]), wide_j)
            return sup + jnp.dot(keep_ref[i], o,
                                 preferred_element_type=jnp.float32)

        sup = lax.fori_loop(0, j, cross, jnp.zeros((1, _B), jnp.float32))
        m0 = jnp.where(sup > 0.0, 0.0, valid_j)

        o_jj = _overlap(_corners_tall(bt_ref[j]), wide_j) * row_lt_col

        def cond(c):
            return c[1]

        def body(c):
            k, _ = c
            s = jnp.dot(k, o_jj, preferred_element_type=jnp.float32)
            k2 = jnp.where(s > 0.0, 0.0, m0)
            return k2, jnp.any(k2 != k)

        k_fix, _ = lax.while_loop(cond, body, (m0, True))
        keep_ref[j] = k_fix
        return 0

    lax.fori_loop(0, nb, outer, 0)

    for j in range(_NB):
        x1, y1, x2, y2, _ = _corners_wide(bw_ref[j])
        row = jnp.concatenate([x1, y1, x2, y2, sw_ref[j]], axis=0)
        out_ref[j] = row * keep_ref[j]


@jax.jit
def kernel(boxes, scores):
    neg, cx, cy, w, h = jax.lax.sort(
        (-scores, boxes[:, 0], boxes[:, 1], boxes[:, 2], boxes[:, 3]),
        dimension=0, is_stable=True, num_keys=1)
    ss = -neg
    sb = jnp.stack([cx, cy, w, h], axis=1)

    sb = jnp.concatenate(
        [sb, jnp.zeros((_NPAD - _N, 4), jnp.float32)], axis=0)
    ss = jnp.concatenate(
        [ss, jnp.full((_NPAD - _N,), -1.0, jnp.float32)], axis=0)

    bt = sb.reshape(_NB, _B, 4)
    bw = jnp.transpose(bt, (0, 2, 1))
    sw = ss.reshape(_NB, 1, _B)

    ow = pl.pallas_call(
        _nms_body,
        out_shape=jax.ShapeDtypeStruct((_NB, 5, _B), jnp.float32),
        scratch_shapes=[pltpu.VMEM((_NB, 1, _B), jnp.float32)],
    )(bt, bw, sw)

    out = jnp.transpose(ow, (1, 0, 2)).reshape(5, _NPAD)[:, :_N]
    return out.T
